# baseline (speedup 1.0000x reference)
import functools

import jax
import jax.numpy as jnp
from jax import lax
from jax.experimental import pallas as pl
from jax.experimental.pallas import tpu as pltpu
from jax.experimental.pallas import tpu_sc as plsc

_NC = 2
_NS = 16
_NW = _NC * _NS


def kernel(token_ids, embed_W, proj_W, scale):
    n = token_ids.size
    bpw = n // _NW
    mesh = plsc.VectorSubcoreMesh(core_axis_name="c", subcore_axis_name="s")

    @functools.partial(
        pl.kernel,
        mesh=mesh,
        out_type=(
            jax.ShapeDtypeStruct((n, 128), jnp.float32),
            jax.ShapeDtypeStruct((n,), jnp.int32),
        ),
        scratch_types=[
            pltpu.VMEM((bpw,), jnp.int32),
            pltpu.VMEM((bpw, 128), jnp.float32),
        ],
    )
    def k(t_hbm, wide_hbm, sel_hbm, tok_v, rows_v):
        wid = lax.axis_index("s") * _NC + lax.axis_index("c")
        base = wid * bpw
        pltpu.sync_copy(t_hbm.at[pl.ds(base, bpw)], tok_v)
        pltpu.sync_copy(rows_v, wide_hbm.at[pl.ds(base, bpw)])
        pltpu.sync_copy(tok_v, sel_hbm.at[pl.ds(base, bpw)])

    return k(token_ids.reshape(-1))
